# Initial kernel scaffold; baseline (speedup 1.0000x reference)
import jax
import jax.numpy as jnp
from jax.experimental import pallas as pl


def kernel(x, token_table, ac, lt_w, lt_b, df_w, df_b):
    raise NotImplementedError("write your pallas kernel here")


# speedup vs baseline: 4.5274x; 4.5274x over previous
import math

import jax
import jax.numpy as jnp
from jax import lax
from jax.experimental import pallas as pl
from jax.experimental.pallas import tpu as pltpu
from jax.experimental.pallas import tpu_sc as plsc

V = 100000
D = 64
W = 80
NC, NS, LN = 2, 16, 16
NW = NC * NS
CH = 128


def _ext_table_body(tok_ref, ac_ref, bias_ref, out_ref):
    i = pl.program_id(0)
    r = tok_ref.shape[0]
    row = (lax.broadcasted_iota(jnp.int32, (r, D), 0) + i * r
           ).astype(jnp.float32)
    col = lax.broadcasted_iota(jnp.int32, (r, D), 1)
    k = ((col // 2) * 2).astype(jnp.float32)
    ang = row * jnp.exp(k * (-math.log(10000.0) / D))
    pos = jnp.where(col % 2 == 0, jnp.sin(ang), jnp.cos(ang))
    out_ref[:, 0:D] = tok_ref[...] + pos + bias_ref[...]
    out_ref[:, D:W] = jnp.broadcast_to(ac_ref[...], (r, W - D))


def _build_ext_table(token_table, ac, bias2d):
    R = 2000
    return pl.pallas_call(
        _ext_table_body,
        grid=(V // R,),
        in_specs=[
            pl.BlockSpec((R, D), lambda i: (i, 0)),
            pl.BlockSpec((R, 1), lambda i: (i, 0)),
            pl.BlockSpec((1, D), lambda i: (0, 0)),
        ],
        out_specs=pl.BlockSpec((R, W), lambda i: (i, 0)),
        out_shape=jax.ShapeDtypeStruct((V, W), jnp.float32),
    )(token_table, ac, bias2d)


def _diff_body(ts_ref, out_ref):
    t = ts_ref[...].astype(jnp.float32)
    prev = jnp.concatenate([t[:, :1], t[:, :-1]], axis=1)
    out_ref[...] = t - prev


def _build_diff(ts2d):
    B, L = ts2d.shape
    R = 512
    return pl.pallas_call(
        _diff_body,
        grid=(B // R,),
        in_specs=[pl.BlockSpec((R, L), lambda i: (i, 0))],
        out_specs=pl.BlockSpec((R, L), lambda i: (i, 0)),
        out_shape=jax.ShapeDtypeStruct((B, L), jnp.float32),
    )(ts2d)


def _sc_body(acts_hbm, diff_hbm, f0_hbm, f1_hbm, ext_hbm, wts_hbm, out_hbm,
             idx_v, d_v, f0_v, f1_v, rows_v, out_v, w_v, sem):
    wid = lax.axis_index("s") * NC + lax.axis_index("c")
    rows_per_w = out_hbm.shape[0] // NW
    base = wid * rows_per_w
    nchunks = rows_per_w // CH

    pltpu.sync_copy(wts_hbm, w_v)
    wlt = [w_v[pl.ds(j * LN, LN)] for j in range(4)]
    w0 = [w_v[pl.ds(D + j * LN, LN)] for j in range(4)]
    w1 = [w_v[pl.ds(2 * D + j * LN, LN)] for j in range(4)]

    def chunk_body(c, tok):
        rbase = base + c * CH
        cps = [pltpu.async_copy(h.at[pl.ds(rbase, CH)], v, sem)
               for h, v in ((acts_hbm, idx_v), (diff_hbm, d_v),
                            (f0_hbm, f0_v), (f1_hbm, f1_v))]
        for cp in cps:
            cp.wait()
        pltpu.async_copy(ext_hbm.at[idx_v], rows_v, sem).wait()

        def group_body(g, tk):
            gb = g * LN
            dv = d_v[pl.ds(gb, LN)]
            f0g = f0_v[pl.ds(gb, LN)].astype(jnp.float32)
            f1g = f1_v[pl.ds(gb, LN)].astype(jnp.float32)
            for r in range(LN):
                i = gb + r
                acv = rows_v[i, pl.ds(D, LN)]
                sb = jnp.full((LN,), acv[0] * dv[r], jnp.float32)
                f0b = jnp.full((LN,), f0g[r], jnp.float32)
                f1b = jnp.full((LN,), f1g[r], jnp.float32)
                for j in range(4):
                    out_v[i, pl.ds(j * LN, LN)] = (
                        rows_v[i, pl.ds(j * LN, LN)]
                        + sb * wlt[j] + f0b * w0[j] + f1b * w1[j])
            return tk

        lax.fori_loop(0, CH // LN, group_body, 0)
        pltpu.sync_copy(out_v, out_hbm.at[pl.ds(rbase, CH), :])
        return tok

    lax.fori_loop(0, nchunks, chunk_body, 0)


def _run_sc(acts, dif, f0r, f1r, ext, wts, n_rows):
    mesh = plsc.VectorSubcoreMesh(core_axis_name="c", subcore_axis_name="s")
    f = pl.kernel(
        _sc_body,
        out_type=jax.ShapeDtypeStruct((n_rows, D), jnp.float32),
        mesh=mesh,
        compiler_params=pltpu.CompilerParams(use_tc_tiling_on_sc=False),
        scratch_types=[
            pltpu.VMEM((CH,), jnp.int32),
            pltpu.VMEM((CH,), jnp.float32),
            pltpu.VMEM((CH,), jnp.int32),
            pltpu.VMEM((CH,), jnp.int32),
            pltpu.VMEM((CH, W), jnp.float32),
            pltpu.VMEM((CH, D), jnp.float32),
            pltpu.VMEM((3 * D,), jnp.float32),
            pltpu.SemaphoreType.DMA,
        ],
    )
    return f(acts, dif, f0r, f1r, ext, wts)


def kernel(x, token_table, ac, lt_w, lt_b, df_w, df_b):
    B, L, _ = x.shape
    n = B * L
    acts = x[:, :, 0].reshape(n)
    f0r = x[:, :, 2].reshape(n)
    f1r = x[:, :, 3].reshape(n)
    bias2d = (lt_b + df_b)[None, :]
    wts = jnp.concatenate([lt_w[:, 0], df_w[:, 0], df_w[:, 1]])
    ext = _build_ext_table(token_table, ac, bias2d)
    dif = _build_diff(x[:, :, 1]).reshape(n)
    out = _run_sc(acts, dif, f0r, f1r, ext, wts, n)
    return out.reshape(B, L, D)


# speedup vs baseline: 5.4176x; 1.1966x over previous
import math

import jax
import jax.numpy as jnp
from jax import lax
from jax.experimental import pallas as pl
from jax.experimental.pallas import tpu as pltpu
from jax.experimental.pallas import tpu_sc as plsc

V = 100000
D = 64
W = 80
NC, NS, LN = 2, 16, 16
NW = NC * NS
CH = 128
NBUF = 4


def _ext_table_body(tok_ref, ac_ref, bias_ref, out_ref):
    i = pl.program_id(0)
    r = tok_ref.shape[0]
    row = (lax.broadcasted_iota(jnp.int32, (r, D), 0) + i * r
           ).astype(jnp.float32)
    col = lax.broadcasted_iota(jnp.int32, (r, D), 1)
    k = ((col // 2) * 2).astype(jnp.float32)
    ang = row * jnp.exp(k * (-math.log(10000.0) / D))
    pos = jnp.where(col % 2 == 0, jnp.sin(ang), jnp.cos(ang))
    out_ref[:, 0:D] = tok_ref[...] + pos + bias_ref[...]
    out_ref[:, D:W] = jnp.broadcast_to(ac_ref[...], (r, W - D))


def _build_ext_table(token_table, ac, bias2d):
    R = 2000
    return pl.pallas_call(
        _ext_table_body,
        grid=(V // R,),
        in_specs=[
            pl.BlockSpec((R, D), lambda i: (i, 0)),
            pl.BlockSpec((R, 1), lambda i: (i, 0)),
            pl.BlockSpec((1, D), lambda i: (0, 0)),
        ],
        out_specs=pl.BlockSpec((R, W), lambda i: (i, 0)),
        out_shape=jax.ShapeDtypeStruct((V, W), jnp.float32),
    )(token_table, ac, bias2d)


def _diff_body(ts_ref, out_ref):
    t = ts_ref[...]
    prev = jnp.concatenate([t[:, :1], t[:, :-1]], axis=1)
    out_ref[...] = t - prev


def _build_diff(ts2d):
    B, L = ts2d.shape
    R = 512
    return pl.pallas_call(
        _diff_body,
        grid=(B // R,),
        in_specs=[pl.BlockSpec((R, L), lambda i: (i, 0))],
        out_specs=pl.BlockSpec((R, L), lambda i: (i, 0)),
        out_shape=jax.ShapeDtypeStruct((B, L), jnp.int32),
    )(ts2d)


def _sc_body(pk_hbm, ext_hbm, wts_hbm, out_hbm,
             in0, in1, in2, in3, r0, r1, r2, r3, o0, o1, o2, o3,
             w_v, in_sem, g_sem, out_sem):
    in_v = [in0, in1, in2, in3]
    rows_v = [r0, r1, r2, r3]
    out_v = [o0, o1, o2, o3]
    wid = lax.axis_index("s") * NC + lax.axis_index("c")
    n_rows = out_hbm.shape[0]
    rows_per_w = n_rows // NW
    base = wid * rows_per_w
    nch = rows_per_w // CH
    cbase = wid * nch

    pltpu.sync_copy(wts_hbm, w_v)
    wlt = [w_v[pl.ds(j * LN, LN)] for j in range(4)]
    w0 = [w_v[pl.ds(D + j * LN, LN)] for j in range(4)]
    w1 = [w_v[pl.ds(2 * D + j * LN, LN)] for j in range(4)]

    def start_in(c, b):
        cc = cbase + jnp.minimum(c, nch - 1)
        pltpu.async_copy(pk_hbm.at[cc], in_v[b], in_sem)

    def wait_in(b):
        pltpu.make_async_copy(pk_hbm.at[cbase], in_v[b], in_sem).wait()

    def start_gather(b):
        pltpu.async_copy(ext_hbm.at[in_v[b].at[0]], rows_v[b], g_sem)

    def wait_gather(b):
        pltpu.make_async_copy(ext_hbm.at[in_v[b].at[0]], rows_v[b],
                              g_sem).wait()

    def start_out(c, b):
        pltpu.async_copy(out_v[b],
                         out_hbm.at[pl.ds(base + c * CH, CH), :], out_sem)

    def drain_out(b):
        pltpu.make_async_copy(out_v[b], out_hbm.at[pl.ds(base, CH), :],
                              out_sem).wait()

    def compute(b):
        inb, rowsb, outb = in_v[b], rows_v[b], out_v[b]

        def group_body(g, tk):
            gb = g * LN
            dv = inb[1, pl.ds(gb, LN)].astype(jnp.float32)
            f0g = inb[2, pl.ds(gb, LN)].astype(jnp.float32)
            f1g = inb[3, pl.ds(gb, LN)].astype(jnp.float32)
            for r in range(LN):
                i = gb + r
                acv = rowsb[i, pl.ds(D, LN)]
                sb = jnp.full((LN,), acv[0] * dv[r], jnp.float32)
                f0b = jnp.full((LN,), f0g[r], jnp.float32)
                f1b = jnp.full((LN,), f1g[r], jnp.float32)
                for j in range(4):
                    outb[i, pl.ds(j * LN, LN)] = (
                        rowsb[i, pl.ds(j * LN, LN)]
                        + sb * wlt[j] + f0b * w0[j] + f1b * w1[j])
            return tk

        lax.fori_loop(0, CH // LN, group_body, 0)

    def step(c, b, drain):
        nb1, nb2 = (b + 1) % NBUF, (b + 2) % NBUF
        wait_in(nb1)
        start_gather(nb1)
        start_in(c + 2, nb2)
        wait_gather(b)
        if drain:
            drain_out(b)
        compute(b)
        start_out(c, b)

    start_in(0, 0)
    wait_in(0)
    start_gather(0)
    start_in(1, 1)
    for b in range(NBUF):
        step(b, b, False)

    def outer(p, tk):
        for b in range(NBUF):
            step(p * NBUF + b, b, True)
        return tk

    lax.fori_loop(1, nch // NBUF, outer, 0)

    wait_in(1)
    wait_gather(0)
    for b in range(NBUF):
        drain_out(b)


def _run_sc(pk, ext, wts, n_rows):
    mesh = plsc.VectorSubcoreMesh(core_axis_name="c", subcore_axis_name="s")
    f = pl.kernel(
        _sc_body,
        out_type=jax.ShapeDtypeStruct((n_rows, D), jnp.float32),
        mesh=mesh,
        compiler_params=pltpu.CompilerParams(use_tc_tiling_on_sc=False),
        scratch_types=(
            [pltpu.VMEM((4, CH), jnp.int32) for _ in range(NBUF)]
            + [pltpu.VMEM((CH, W), jnp.float32) for _ in range(NBUF)]
            + [pltpu.VMEM((CH, D), jnp.float32) for _ in range(NBUF)]
            + [pltpu.VMEM((3 * D,), jnp.float32),
               pltpu.SemaphoreType.DMA,
               pltpu.SemaphoreType.DMA,
               pltpu.SemaphoreType.DMA]
        ),
    )
    return f(pk, ext, wts)


def kernel(x, token_table, ac, lt_w, lt_b, df_w, df_b):
    B, L, _ = x.shape
    n = B * L
    acts = x[:, :, 0].reshape(n)
    f0r = x[:, :, 2].reshape(n)
    f1r = x[:, :, 3].reshape(n)
    bias2d = (lt_b + df_b)[None, :]
    wts = jnp.concatenate([lt_w[:, 0], df_w[:, 0], df_w[:, 1]])
    ext = _build_ext_table(token_table, ac, bias2d)
    dif = _build_diff(x[:, :, 1]).reshape(n)
    pk = (jnp.stack([acts, dif, f0r, f1r])
          .reshape(4, n // CH, CH).transpose(1, 0, 2))
    out = _run_sc(pk, ext, wts, n)
    return out.reshape(B, L, D)


# speedup vs baseline: 6.1393x; 1.1332x over previous
import math

import jax
import jax.numpy as jnp
from jax import lax
from jax.experimental import pallas as pl
from jax.experimental.pallas import tpu as pltpu
from jax.experimental.pallas import tpu_sc as plsc

V = 100000
D = 64
W = 80
NC, NS, LN = 2, 16, 16
NW = NC * NS
CH = 128
NBUF = 4


def _ext_table_body(tok_ref, ac_ref, bias_ref, out_ref):
    i = pl.program_id(0)
    r = tok_ref.shape[0]
    row = (lax.broadcasted_iota(jnp.int32, (r, D), 0) + i * r
           ).astype(jnp.float32)
    col = lax.broadcasted_iota(jnp.int32, (r, D), 1)
    k = ((col // 2) * 2).astype(jnp.float32)
    ang = row * jnp.exp(k * (-math.log(10000.0) / D))
    pos = jnp.where(col % 2 == 0, jnp.sin(ang), jnp.cos(ang))
    out_ref[:, 0:D] = tok_ref[...] + pos + bias_ref[...]
    out_ref[:, D:W] = jnp.broadcast_to(ac_ref[...], (r, W - D))


def _build_ext_table(token_table, ac, bias2d):
    R = 2000
    return pl.pallas_call(
        _ext_table_body,
        grid=(V // R,),
        in_specs=[
            pl.BlockSpec((R, D), lambda i: (i, 0)),
            pl.BlockSpec((R, 1), lambda i: (i, 0)),
            pl.BlockSpec((1, D), lambda i: (0, 0)),
        ],
        out_specs=pl.BlockSpec((R, W), lambda i: (i, 0)),
        out_shape=jax.ShapeDtypeStruct((V, W), jnp.float32),
    )(token_table, ac, bias2d)


def _diff_body(ts_ref, out_ref):
    t = ts_ref[...]
    prev = jnp.concatenate([t[:, :1], t[:, :-1]], axis=1)
    out_ref[...] = t - prev


def _build_diff(ts2d):
    B, L = ts2d.shape
    R = 512
    return pl.pallas_call(
        _diff_body,
        grid=(B // R,),
        in_specs=[pl.BlockSpec((R, L), lambda i: (i, 0))],
        out_specs=pl.BlockSpec((R, L), lambda i: (i, 0)),
        out_shape=jax.ShapeDtypeStruct((B, L), jnp.int32),
    )(ts2d)


def _sc_body(pk_hbm, ext_hbm, wts_hbm, out_hbm,
             in0, in1, in2, in3, r0, r1, r2, r3, o0, o1, o2, o3,
             w_v, in_sem, g_sem, out_sem):
    in_v = [in0, in1, in2, in3]
    rows_v = [r0, r1, r2, r3]
    out_v = [o0, o1, o2, o3]
    wid = lax.axis_index("s") * NC + lax.axis_index("c")
    n_rows = out_hbm.shape[0]
    rows_per_w = n_rows // NW
    base = wid * rows_per_w
    nch = rows_per_w // CH

    pltpu.sync_copy(wts_hbm, w_v)
    wlt = [w_v[pl.ds(j * LN, LN)] for j in range(4)]
    w0 = [w_v[pl.ds(D + j * LN, LN)] for j in range(4)]
    w1 = [w_v[pl.ds(2 * D + j * LN, LN)] for j in range(4)]

    def start_in(c, b):
        rb = base + jnp.minimum(c, nch - 1) * CH
        pltpu.async_copy(pk_hbm.at[:, pl.ds(rb, CH)], in_v[b], in_sem)

    def wait_in(b):
        pltpu.make_async_copy(pk_hbm.at[:, pl.ds(base, CH)], in_v[b],
                              in_sem).wait()

    def start_gather(b):
        pltpu.async_copy(ext_hbm.at[in_v[b].at[0]], rows_v[b], g_sem)

    def wait_gather(b):
        pltpu.make_async_copy(ext_hbm.at[in_v[b].at[0]], rows_v[b],
                              g_sem).wait()

    def start_out(c, b):
        pltpu.async_copy(out_v[b],
                         out_hbm.at[pl.ds(base + c * CH, CH), :], out_sem)

    def drain_out(b):
        pltpu.make_async_copy(out_v[b], out_hbm.at[pl.ds(base, CH), :],
                              out_sem).wait()

    def compute(b):
        inb, rowsb, outb = in_v[b], rows_v[b], out_v[b]

        def group_body(g, tk):
            gb = g * LN
            dv = inb[1, pl.ds(gb, LN)].astype(jnp.float32)
            f0g = inb[2, pl.ds(gb, LN)].astype(jnp.float32)
            f1g = inb[3, pl.ds(gb, LN)].astype(jnp.float32)
            for r in range(LN):
                i = gb + r
                rsel = jnp.full((LN,), r, jnp.int32)
                sb = rowsb[i, pl.ds(D, LN)] * jnp.take(dv, rsel)
                f0b = jnp.take(f0g, rsel)
                f1b = jnp.take(f1g, rsel)
                for j in range(4):
                    outb[i, pl.ds(j * LN, LN)] = (
                        rowsb[i, pl.ds(j * LN, LN)]
                        + sb * wlt[j] + f0b * w0[j] + f1b * w1[j])
            return tk

        lax.fori_loop(0, CH // LN, group_body, 0)

    def step(c, b, drain):
        nb1, nb2 = (b + 1) % NBUF, (b + 2) % NBUF
        wait_in(nb1)
        start_gather(nb1)
        start_in(c + 2, nb2)
        wait_gather(b)
        if drain:
            drain_out(b)
        compute(b)
        start_out(c, b)

    start_in(0, 0)
    wait_in(0)
    start_gather(0)
    start_in(1, 1)
    for b in range(NBUF):
        step(b, b, False)

    def outer(p, tk):
        for b in range(NBUF):
            step(p * NBUF + b, b, True)
        return tk

    lax.fori_loop(1, nch // NBUF, outer, 0)

    wait_in(1)
    wait_gather(0)
    for b in range(NBUF):
        drain_out(b)


def _run_sc(pk, ext, wts, n_rows):
    mesh = plsc.VectorSubcoreMesh(core_axis_name="c", subcore_axis_name="s")
    f = pl.kernel(
        _sc_body,
        out_type=jax.ShapeDtypeStruct((n_rows, D), jnp.float32),
        mesh=mesh,
        compiler_params=pltpu.CompilerParams(use_tc_tiling_on_sc=False),
        scratch_types=(
            [pltpu.VMEM((4, CH), jnp.int32) for _ in range(NBUF)]
            + [pltpu.VMEM((CH, W), jnp.float32) for _ in range(NBUF)]
            + [pltpu.VMEM((CH, D), jnp.float32) for _ in range(NBUF)]
            + [pltpu.VMEM((3 * D,), jnp.float32),
               pltpu.SemaphoreType.DMA,
               pltpu.SemaphoreType.DMA,
               pltpu.SemaphoreType.DMA]
        ),
    )
    return f(pk, ext, wts)


def kernel(x, token_table, ac, lt_w, lt_b, df_w, df_b):
    B, L, _ = x.shape
    n = B * L
    acts = x[:, :, 0].reshape(n)
    f0r = x[:, :, 2].reshape(n)
    f1r = x[:, :, 3].reshape(n)
    bias2d = (lt_b + df_b)[None, :]
    wts = jnp.concatenate([lt_w[:, 0], df_w[:, 0], df_w[:, 1]])
    ext = _build_ext_table(token_table, ac, bias2d)
    dif = _build_diff(x[:, :, 1]).reshape(n)
    pk = jnp.stack([acts, dif, f0r, f1r])
    out = _run_sc(pk, ext, wts, n)
    return out.reshape(B, L, D)


# speedup vs baseline: 8.6282x; 1.4054x over previous
import math

import jax
import jax.numpy as jnp
from jax import lax
from jax.experimental import pallas as pl
from jax.experimental.pallas import tpu as pltpu
from jax.experimental.pallas import tpu_sc as plsc

V = 100000
D = 64
W = 80
NC, NS, LN = 2, 16, 16
NW = NC * NS
CH = 128
NBUF = 4


def _ext_table_body(tok_ref, ac_ref, bias_ref, out_ref):
    i = pl.program_id(0)
    r = tok_ref.shape[0]
    row = (lax.broadcasted_iota(jnp.int32, (r, D), 0) + i * r
           ).astype(jnp.float32)
    col = lax.broadcasted_iota(jnp.int32, (r, D), 1)
    k = ((col // 2) * 2).astype(jnp.float32)
    ang = row * jnp.exp(k * (-math.log(10000.0) / D))
    pos = jnp.where(col % 2 == 0, jnp.sin(ang), jnp.cos(ang))
    out_ref[:, 0:D] = tok_ref[...] + pos + bias_ref[...]
    out_ref[:, D:W] = jnp.broadcast_to(ac_ref[...], (r, W - D))


def _build_ext_table(token_table, ac, bias2d):
    R = 2000
    return pl.pallas_call(
        _ext_table_body,
        grid=(V // R,),
        in_specs=[
            pl.BlockSpec((R, D), lambda i: (i, 0)),
            pl.BlockSpec((R, 1), lambda i: (i, 0)),
            pl.BlockSpec((1, D), lambda i: (0, 0)),
        ],
        out_specs=pl.BlockSpec((R, W), lambda i: (i, 0)),
        out_shape=jax.ShapeDtypeStruct((V, W), jnp.float32),
    )(token_table, ac, bias2d)


def _diff_body(ts_ref, out_ref):
    t = ts_ref[...]
    prev = jnp.concatenate([t[:, :1], t[:, :-1]], axis=1)
    out_ref[...] = t - prev


def _build_diff(ts2d):
    B, L = ts2d.shape
    R = 512
    return pl.pallas_call(
        _diff_body,
        grid=(B // R,),
        in_specs=[pl.BlockSpec((R, L), lambda i: (i, 0))],
        out_specs=pl.BlockSpec((R, L), lambda i: (i, 0)),
        out_shape=jax.ShapeDtypeStruct((B, L), jnp.int32),
    )(ts2d)


def _sc_body(pk_hbm, ext_hbm, wts_hbm, out_hbm,
             in0, in1, in2, in3, r0, r1, r2, r3, o0, o1, o2, o3,
             w_v, in_sem, g_sem, out_sem):
    in_v = [in0, in1, in2, in3]
    rows_v = [r0, r1, r2, r3]
    out_v = [o0, o1, o2, o3]
    wid = lax.axis_index("s") * NC + lax.axis_index("c")
    n_rows = out_hbm.shape[0]
    rows_per_w = n_rows // NW
    base = wid * rows_per_w
    nch = rows_per_w // CH

    pltpu.sync_copy(wts_hbm, w_v)
    wlt = [w_v[pl.ds(j * LN, LN)] for j in range(4)]
    w0 = [w_v[pl.ds(D + j * LN, LN)] for j in range(4)]
    w1 = [w_v[pl.ds(2 * D + j * LN, LN)] for j in range(4)]

    def start_in(c, b):
        rb = base + jnp.minimum(c, nch - 1) * CH
        pltpu.async_copy(pk_hbm.at[:, pl.ds(rb, CH)], in_v[b], in_sem)

    def wait_in(b):
        pltpu.make_async_copy(pk_hbm.at[:, pl.ds(base, CH)], in_v[b],
                              in_sem).wait()

    def start_gather(b):
        pltpu.async_copy(ext_hbm.at[in_v[b].at[0]], rows_v[b], g_sem)

    def wait_gather(b):
        pltpu.make_async_copy(ext_hbm.at[in_v[b].at[0]], rows_v[b],
                              g_sem).wait()

    def start_out(c, b):
        pltpu.async_copy(out_v[b],
                         out_hbm.at[pl.ds(base + c * CH, CH), :], out_sem)

    def drain_out(b):
        pltpu.make_async_copy(out_v[b], out_hbm.at[pl.ds(base, CH), :],
                              out_sem).wait()

    def compute(b):
        inb, rowsb, outb = in_v[b], rows_v[b], out_v[b]

        def group_body(g, tk):
            gb = g * LN
            dv = inb[1, pl.ds(gb, LN)].astype(jnp.float32)
            f0g = inb[2, pl.ds(gb, LN)].astype(jnp.float32)
            f1g = inb[3, pl.ds(gb, LN)].astype(jnp.float32)
            for rr in range(0, LN, 2):
                rows = []
                for r in (rr, rr + 1):
                    i = gb + r
                    rsel = jnp.full((LN,), r, jnp.int32)
                    rj = [rowsb[i, pl.ds(j * LN, LN)] for j in range(4)]
                    sb = rowsb[i, pl.ds(D, LN)] * jnp.take(dv, rsel)
                    rows.append((i, rj, sb, jnp.take(f0g, rsel),
                                 jnp.take(f1g, rsel)))
                for j in range(4):
                    for i, rj, sb, f0b, f1b in rows:
                        outb[i, pl.ds(j * LN, LN)] = (
                            (rj[j] + sb * wlt[j])
                            + (f0b * w0[j] + f1b * w1[j]))
            return tk

        lax.fori_loop(0, CH // LN, group_body, 0)

    def step(c, b, drain):
        nb1, nb2 = (b + 1) % NBUF, (b + 2) % NBUF
        wait_in(nb1)
        start_gather(nb1)
        start_in(c + 2, nb2)
        wait_gather(b)
        if drain:
            drain_out(b)
        compute(b)
        start_out(c, b)

    start_in(0, 0)
    wait_in(0)
    start_gather(0)
    start_in(1, 1)
    for b in range(NBUF):
        step(b, b, False)

    def outer(p, tk):
        for b in range(NBUF):
            step(p * NBUF + b, b, True)
        return tk

    lax.fori_loop(1, nch // NBUF, outer, 0)

    wait_in(1)
    wait_gather(0)
    for b in range(NBUF):
        drain_out(b)


def _run_sc(pk, ext, wts, n_rows):
    mesh = plsc.VectorSubcoreMesh(core_axis_name="c", subcore_axis_name="s")
    f = pl.kernel(
        _sc_body,
        out_type=jax.ShapeDtypeStruct((n_rows, D), jnp.float32),
        mesh=mesh,
        compiler_params=pltpu.CompilerParams(use_tc_tiling_on_sc=False),
        scratch_types=(
            [pltpu.VMEM((4, CH), jnp.int32) for _ in range(NBUF)]
            + [pltpu.VMEM((CH, W), jnp.float32) for _ in range(NBUF)]
            + [pltpu.VMEM((CH, D), jnp.float32) for _ in range(NBUF)]
            + [pltpu.VMEM((3 * D,), jnp.float32),
               pltpu.SemaphoreType.DMA,
               pltpu.SemaphoreType.DMA,
               pltpu.SemaphoreType.DMA]
        ),
    )
    return f(pk, ext, wts)


def kernel(x, token_table, ac, lt_w, lt_b, df_w, df_b):
    B, L, _ = x.shape
    n = B * L
    acts = x[:, :, 0].reshape(n)
    f0r = x[:, :, 2].reshape(n)
    f1r = x[:, :, 3].reshape(n)
    bias2d = (lt_b + df_b)[None, :]
    wts = jnp.concatenate([lt_w[:, 0], df_w[:, 0], df_w[:, 1]])
    ext = _build_ext_table(token_table, ac, bias2d)
    dif = _build_diff(x[:, :, 1]).reshape(n)
    pk = jnp.stack([acts, dif, f0r, f1r])
    out = _run_sc(pk, ext, wts, n)
    return out.reshape(B, L, D)


# speedup vs baseline: 12.0839x; 1.4005x over previous
import math

import jax
import jax.numpy as jnp
from jax import lax
from jax.experimental import pallas as pl
from jax.experimental.pallas import tpu as pltpu
from jax.experimental.pallas import tpu_sc as plsc

V = 100000
D = 64
W = 128
NC, NS, LN = 2, 16, 16
NW = NC * NS
CH = 128
NBUF = 4


def _ext_table_body(tok_ref, ac_ref, bias_ref, out_ref):
    i = pl.program_id(0)
    r = tok_ref.shape[0]
    row = (lax.broadcasted_iota(jnp.int32, (r, D), 0) + i * r
           ).astype(jnp.float32)
    col = lax.broadcasted_iota(jnp.int32, (r, D), 1)
    k = ((col // 2) * 2).astype(jnp.float32)
    ang = row * jnp.exp(k * (-math.log(10000.0) / D))
    pos = jnp.where(col % 2 == 0, jnp.sin(ang), jnp.cos(ang))
    out_ref[:, 0:D] = tok_ref[...] + pos + bias_ref[...]
    out_ref[:, D:W] = jnp.broadcast_to(ac_ref[...], (r, W - D))


def _build_ext_table(token_table, ac, bias2d):
    R = 2000
    return pl.pallas_call(
        _ext_table_body,
        grid=(V // R,),
        in_specs=[
            pl.BlockSpec((R, D), lambda i: (i, 0)),
            pl.BlockSpec((R, 1), lambda i: (i, 0)),
            pl.BlockSpec((1, D), lambda i: (0, 0)),
        ],
        out_specs=pl.BlockSpec((R, W), lambda i: (i, 0)),
        out_shape=jax.ShapeDtypeStruct((V, W), jnp.float32),
    )(token_table, ac, bias2d)


def _diff_body(ts_ref, out_ref):
    t = ts_ref[...]
    prev = jnp.concatenate([t[:, :1], t[:, :-1]], axis=1)
    out_ref[...] = t - prev


def _build_diff(ts2d):
    B, L = ts2d.shape
    R = 512
    return pl.pallas_call(
        _diff_body,
        grid=(B // R,),
        in_specs=[pl.BlockSpec((R, L), lambda i: (i, 0))],
        out_specs=pl.BlockSpec((R, L), lambda i: (i, 0)),
        out_shape=jax.ShapeDtypeStruct((B, L), jnp.int32),
    )(ts2d)


def _sc_body(pk_hbm, ext_hbm, wts_hbm, out_hbm,
             in0, in1, in2, in3, r0, r1, o0, o1,
             w_v, in_sem, g_sem, out_sem):
    in_v = [in0, in1, in2, in3]
    rows_v = [r0, r1]
    out_v = [o0, o1]
    wid = lax.axis_index("s") * NC + lax.axis_index("c")
    n_rows = out_hbm.shape[0]
    rows_per_w = n_rows // NW
    base = wid * rows_per_w
    nch = rows_per_w // CH

    pltpu.sync_copy(wts_hbm, w_v)
    wlt = [w_v[pl.ds(j * LN, LN)] for j in range(4)]
    w0 = [w_v[pl.ds(D + j * LN, LN)] for j in range(4)]
    w1 = [w_v[pl.ds(2 * D + j * LN, LN)] for j in range(4)]

    def start_in(c, b):
        rb = base + jnp.minimum(c, nch - 1) * CH
        pltpu.async_copy(pk_hbm.at[:, pl.ds(rb, CH)], in_v[b], in_sem)

    def wait_in(b):
        pltpu.make_async_copy(pk_hbm.at[:, pl.ds(base, CH)], in_v[b],
                              in_sem).wait()

    def start_gather(bi, br):
        pltpu.async_copy(ext_hbm.at[in_v[bi].at[0]], rows_v[br], g_sem)

    def wait_gather(bi, br):
        pltpu.make_async_copy(ext_hbm.at[in_v[bi].at[0]], rows_v[br],
                              g_sem).wait()

    def start_out(c, br):
        pltpu.async_copy(out_v[br],
                         out_hbm.at[pl.ds(base + c * CH, CH), :], out_sem)

    def drain_out(br):
        pltpu.make_async_copy(out_v[br], out_hbm.at[pl.ds(base, CH), :],
                              out_sem).wait()

    def compute(b, br):
        inb, rowsb, outb = in_v[b], rows_v[br], out_v[br]

        def group_body(g, tk):
            gb = g * LN
            dv = inb[1, pl.ds(gb, LN)].astype(jnp.float32)
            f0g = inb[2, pl.ds(gb, LN)].astype(jnp.float32)
            f1g = inb[3, pl.ds(gb, LN)].astype(jnp.float32)
            for rr in range(0, LN, 2):
                rows = []
                for r in (rr, rr + 1):
                    i = gb + r
                    rsel = jnp.full((LN,), r, jnp.int32)
                    rj = [rowsb[i, pl.ds(j * LN, LN)] for j in range(4)]
                    sb = rowsb[i, pl.ds(D, LN)] * jnp.take(dv, rsel)
                    rows.append((i, rj, sb, jnp.take(f0g, rsel),
                                 jnp.take(f1g, rsel)))
                for j in range(4):
                    for i, rj, sb, f0b, f1b in rows:
                        outb[i, pl.ds(j * LN, LN)] = (
                            (rj[j] + sb * wlt[j])
                            + (f0b * w0[j] + f1b * w1[j]))
            return tk

        lax.fori_loop(0, CH // LN, group_body, 0)

    def step(c, b, drain):
        wait_in((b + 1) % NBUF)
        start_gather((b + 1) % NBUF, (b + 1) % 2)
        start_in(c + 2, (b + 2) % NBUF)
        wait_gather((b + 1) % NBUF, b % 2)
        if drain:
            drain_out(b % 2)
        compute(b, b % 2)
        start_out(c, b % 2)

    start_in(0, 0)
    wait_in(0)
    start_gather(0, 0)
    start_in(1, 1)
    for b in range(NBUF):
        step(b, b, b >= 2)

    def outer(p, tk):
        for b in range(NBUF):
            step(p * NBUF + b, b, True)
        return tk

    lax.fori_loop(1, nch // NBUF, outer, 0)

    wait_in(1)
    wait_gather(0, 0)
    for b in range(2):
        drain_out(b)


def _run_sc(pk, ext, wts, n_rows):
    mesh = plsc.VectorSubcoreMesh(core_axis_name="c", subcore_axis_name="s")
    f = pl.kernel(
        _sc_body,
        out_type=jax.ShapeDtypeStruct((n_rows, D), jnp.float32),
        mesh=mesh,
        compiler_params=pltpu.CompilerParams(use_tc_tiling_on_sc=True),
        scratch_types=(
            [pltpu.VMEM((4, CH), jnp.int32) for _ in range(NBUF)]
            + [pltpu.VMEM((CH, W), jnp.float32) for _ in range(2)]
            + [pltpu.VMEM((CH, D), jnp.float32) for _ in range(2)]
            + [pltpu.VMEM((3 * D,), jnp.float32),
               pltpu.SemaphoreType.DMA,
               pltpu.SemaphoreType.DMA,
               pltpu.SemaphoreType.DMA]
        ),
    )
    return f(pk, ext, wts)


def kernel(x, token_table, ac, lt_w, lt_b, df_w, df_b):
    B, L, _ = x.shape
    n = B * L
    acts = x[:, :, 0].reshape(n)
    f0r = x[:, :, 2].reshape(n)
    f1r = x[:, :, 3].reshape(n)
    bias2d = (lt_b + df_b)[None, :]
    wts = jnp.concatenate([lt_w[:, 0], df_w[:, 0], df_w[:, 1]])
    ext = _build_ext_table(token_table, ac, bias2d)
    dif = _build_diff(x[:, :, 1]).reshape(n)
    pk = jnp.stack([acts, dif, f0r, f1r])
    out = _run_sc(pk, ext, wts, n)
    return out.reshape(B, L, D)


# speedup vs baseline: 12.0951x; 1.0009x over previous
import math

import jax
import jax.numpy as jnp
from jax import lax
from jax.experimental import pallas as pl
from jax.experimental.pallas import tpu as pltpu
from jax.experimental.pallas import tpu_sc as plsc

V = 100000
D = 64
W = 128
NC, NS, LN = 2, 16, 16
NW = NC * NS
CH = 128
NBUF = 4


def _ext_table_body(tok_ref, ac_ref, bias_ref, out_ref):
    i = pl.program_id(0)
    r = tok_ref.shape[0]
    row = (lax.broadcasted_iota(jnp.int32, (r, D), 0) + i * r
           ).astype(jnp.float32)
    col = lax.broadcasted_iota(jnp.int32, (r, D), 1)
    k = ((col // 2) * 2).astype(jnp.float32)
    ang = row * jnp.exp(k * (-math.log(10000.0) / D))
    pos = jnp.where(col % 2 == 0, jnp.sin(ang), jnp.cos(ang))
    out_ref[:, 0:D] = tok_ref[...] + pos + bias_ref[...]
    out_ref[:, D:W] = jnp.broadcast_to(ac_ref[...], (r, W - D))


def _build_ext_table(token_table, ac, bias2d):
    R = 2000
    return pl.pallas_call(
        _ext_table_body,
        grid=(V // R,),
        in_specs=[
            pl.BlockSpec((R, D), lambda i: (i, 0)),
            pl.BlockSpec((R, 1), lambda i: (i, 0)),
            pl.BlockSpec((1, D), lambda i: (0, 0)),
        ],
        out_specs=pl.BlockSpec((R, W), lambda i: (i, 0)),
        out_shape=jax.ShapeDtypeStruct((V, W), jnp.float32),
    )(token_table, ac, bias2d)


def _diff_body(ts_ref, out_ref):
    t = ts_ref[...]
    prev = jnp.concatenate([t[:, :1], t[:, :-1]], axis=1)
    out_ref[...] = t - prev


def _build_diff(ts2d):
    B, L = ts2d.shape
    R = 512
    return pl.pallas_call(
        _diff_body,
        grid=(B // R,),
        in_specs=[pl.BlockSpec((R, L), lambda i: (i, 0))],
        out_specs=pl.BlockSpec((R, L), lambda i: (i, 0)),
        out_shape=jax.ShapeDtypeStruct((B, L), jnp.int32),
    )(ts2d)


def _sc_body(pk_hbm, ext_hbm, wts_hbm, out_hbm,
             in0, in1, in2, in3, r0, r1, o0, o1,
             w_v, in_sem, g_sem, out_sem):
    in_v = [in0, in1, in2, in3]
    rows_v = [r0, r1]
    out_v = [o0, o1]
    wid = lax.axis_index("s") * NC + lax.axis_index("c")
    n_rows = out_hbm.shape[0]
    rows_per_w = n_rows // NW
    base = wid * rows_per_w
    nch = rows_per_w // CH

    pltpu.sync_copy(wts_hbm, w_v)
    wlt = [w_v[pl.ds(j * LN, LN)] for j in range(4)]
    w0 = [w_v[pl.ds(D + j * LN, LN)] for j in range(4)]
    w1 = [w_v[pl.ds(2 * D + j * LN, LN)] for j in range(4)]

    def start_in(c, b):
        rb = base + jnp.minimum(c, nch - 1) * CH
        pltpu.async_copy(pk_hbm.at[:, pl.ds(rb, CH)], in_v[b], in_sem)

    def wait_in(b):
        pltpu.make_async_copy(pk_hbm.at[:, pl.ds(base, CH)], in_v[b],
                              in_sem).wait()

    def start_gather(bi, br):
        pltpu.async_copy(ext_hbm.at[in_v[bi].at[0]], rows_v[br], g_sem)

    def wait_gather(bi, br):
        pltpu.make_async_copy(ext_hbm.at[in_v[bi].at[0]], rows_v[br],
                              g_sem).wait()

    def start_out(c, br):
        pltpu.async_copy(out_v[br],
                         out_hbm.at[pl.ds(base + c * CH, CH), :], out_sem)

    def drain_out(br):
        pltpu.make_async_copy(out_v[br], out_hbm.at[pl.ds(base, CH), :],
                              out_sem).wait()

    def compute(b, br):
        inb, rowsb, outb = in_v[b], rows_v[br], out_v[br]

        def group_body(g, tk):
            gb = g * LN
            dv = inb[1, pl.ds(gb, LN)].astype(jnp.float32)
            f0g = inb[2, pl.ds(gb, LN)].astype(jnp.float32)
            f1g = inb[3, pl.ds(gb, LN)].astype(jnp.float32)
            for rr in range(0, LN, 2):
                rows = []
                for r in (rr, rr + 1):
                    i = gb + r
                    rsel = jnp.full((LN,), r, jnp.int32)
                    sb = rowsb[i, pl.ds(D, LN)] * jnp.take(dv, rsel)
                    rows.append((i, sb, jnp.take(f0g, rsel),
                                 jnp.take(f1g, rsel)))
                for j in range(4):
                    for i, sb, f0b, f1b in rows:
                        outb[i, pl.ds(j * LN, LN)] = (
                            (rowsb[i, pl.ds(j * LN, LN)] + sb * wlt[j])
                            + (f0b * w0[j] + f1b * w1[j]))
            return tk

        lax.fori_loop(0, CH // LN, group_body, 0)

    def step(c, b, drain):
        wait_in((b + 1) % NBUF)
        start_gather((b + 1) % NBUF, (b + 1) % 2)
        start_in(c + 2, (b + 2) % NBUF)
        wait_gather((b + 1) % NBUF, b % 2)
        if drain:
            drain_out(b % 2)
        compute(b, b % 2)
        start_out(c, b % 2)

    start_in(0, 0)
    wait_in(0)
    start_gather(0, 0)
    start_in(1, 1)
    for b in range(NBUF):
        step(b, b, b >= 2)

    def outer(p, tk):
        for b in range(NBUF):
            step(p * NBUF + b, b, True)
        return tk

    lax.fori_loop(1, nch // NBUF, outer, 0)

    wait_in(1)
    wait_gather(0, 0)
    for b in range(2):
        drain_out(b)


def _run_sc(pk, ext, wts, n_rows):
    mesh = plsc.VectorSubcoreMesh(core_axis_name="c", subcore_axis_name="s")
    f = pl.kernel(
        _sc_body,
        out_type=jax.ShapeDtypeStruct((n_rows, D), jnp.float32),
        mesh=mesh,
        compiler_params=pltpu.CompilerParams(use_tc_tiling_on_sc=True),
        scratch_types=(
            [pltpu.VMEM((4, CH), jnp.int32) for _ in range(NBUF)]
            + [pltpu.VMEM((CH, W), jnp.float32) for _ in range(2)]
            + [pltpu.VMEM((CH, D), jnp.float32) for _ in range(2)]
            + [pltpu.VMEM((3 * D,), jnp.float32),
               pltpu.SemaphoreType.DMA,
               pltpu.SemaphoreType.DMA,
               pltpu.SemaphoreType.DMA]
        ),
    )
    return f(pk, ext, wts)


def kernel(x, token_table, ac, lt_w, lt_b, df_w, df_b):
    B, L, _ = x.shape
    n = B * L
    acts = x[:, :, 0].reshape(n)
    f0r = x[:, :, 2].reshape(n)
    f1r = x[:, :, 3].reshape(n)
    bias2d = (lt_b + df_b)[None, :]
    wts = jnp.concatenate([lt_w[:, 0], df_w[:, 0], df_w[:, 1]])
    ext = _build_ext_table(token_table, ac, bias2d)
    dif = _build_diff(x[:, :, 1]).reshape(n)
    pk = jnp.stack([acts, dif, f0r, f1r])
    out = _run_sc(pk, ext, wts, n)
    return out.reshape(B, L, D)


# speedup vs baseline: 13.8469x; 1.1448x over previous
import math

import jax
import jax.numpy as jnp
from jax import lax
from jax.experimental import pallas as pl
from jax.experimental.pallas import tpu as pltpu
from jax.experimental.pallas import tpu_sc as plsc

V = 100000
D = 64
W = 128
NC, NS, LN = 2, 16, 16
NW = NC * NS
CH = 128
NBUF = 4


_PI_HI = 3.140625
_PI_LO = math.pi - 3.140625
_SINC = (1.0, -1 / 6, 1 / 120, -1 / 5040, 1 / 362880)
_COSC = (1.0, -1 / 2, 1 / 24, -1 / 720, 1 / 40320)


def _ext_table_body(tok_ref, ac_ref, bias_ref, out_ref):
    i = pl.program_id(0)
    r = tok_ref.shape[0]
    row = (lax.broadcasted_iota(jnp.int32, (r, D), 0) + i * r
           ).astype(jnp.float32)
    col = lax.broadcasted_iota(jnp.int32, (r, D), 1)
    k = ((col // 2) * 2).astype(jnp.float32)
    ang = row * jnp.exp(k * (-math.log(10000.0) / D))
    q = jnp.floor(ang * (1.0 / math.pi) + 0.5)
    rr = (ang - q * _PI_HI) - q * _PI_LO
    r2 = rr * rr
    qh = q * 0.5
    sign = 1.0 - 4.0 * (qh - jnp.floor(qh))
    even = col % 2 == 0
    p = jnp.where(even, _SINC[4], _COSC[4])
    for t in range(3, -1, -1):
        p = p * r2 + jnp.where(even, _SINC[t], _COSC[t])
    pos = sign * p * jnp.where(even, rr, 1.0)
    out_ref[:, 0:D] = tok_ref[...] + pos + bias_ref[...]
    out_ref[:, D:W] = jnp.broadcast_to(ac_ref[...], (r, W - D))


def _build_ext_table(token_table, ac, bias2d):
    R = 2000
    return pl.pallas_call(
        _ext_table_body,
        grid=(V // R,),
        in_specs=[
            pl.BlockSpec((R, D), lambda i: (i, 0)),
            pl.BlockSpec((R, 1), lambda i: (i, 0)),
            pl.BlockSpec((1, D), lambda i: (0, 0)),
        ],
        out_specs=pl.BlockSpec((R, W), lambda i: (i, 0)),
        out_shape=jax.ShapeDtypeStruct((V, W), jnp.float32),
    )(token_table, ac, bias2d)


def _diff_body(ts_ref, out_ref):
    t = ts_ref[...]
    prev = jnp.concatenate([t[:, :1], t[:, :-1]], axis=1)
    out_ref[...] = t - prev


def _build_diff(ts2d):
    B, L = ts2d.shape
    R = 512
    return pl.pallas_call(
        _diff_body,
        grid=(B // R,),
        in_specs=[pl.BlockSpec((R, L), lambda i: (i, 0))],
        out_specs=pl.BlockSpec((R, L), lambda i: (i, 0)),
        out_shape=jax.ShapeDtypeStruct((B, L), jnp.int32),
    )(ts2d)


def _sc_body(pk_hbm, ext_hbm, wts_hbm, out_hbm,
             in0, in1, in2, in3, r0, r1, o0, o1,
             w_v, in_sem, g_sem, out_sem):
    in_v = [in0, in1, in2, in3]
    rows_v = [r0, r1]
    out_v = [o0, o1]
    wid = lax.axis_index("s") * NC + lax.axis_index("c")
    n_rows = out_hbm.shape[0]
    rows_per_w = n_rows // NW
    base = wid * rows_per_w
    nch = rows_per_w // CH

    pltpu.sync_copy(wts_hbm, w_v)
    wlt = [w_v[pl.ds(j * LN, LN)] for j in range(4)]
    w0 = [w_v[pl.ds(D + j * LN, LN)] for j in range(4)]
    w1 = [w_v[pl.ds(2 * D + j * LN, LN)] for j in range(4)]

    def start_in(c, b):
        rb = base + jnp.minimum(c, nch - 1) * CH
        pltpu.async_copy(pk_hbm.at[:, pl.ds(rb, CH)], in_v[b], in_sem)

    def wait_in(b):
        pltpu.make_async_copy(pk_hbm.at[:, pl.ds(base, CH)], in_v[b],
                              in_sem).wait()

    def start_gather(bi, br):
        pltpu.async_copy(ext_hbm.at[in_v[bi].at[0]], rows_v[br], g_sem)

    def wait_gather(bi, br):
        pltpu.make_async_copy(ext_hbm.at[in_v[bi].at[0]], rows_v[br],
                              g_sem).wait()

    def start_out(c, br):
        pltpu.async_copy(out_v[br],
                         out_hbm.at[pl.ds(base + c * CH, CH), :], out_sem)

    def drain_out(br):
        pltpu.make_async_copy(out_v[br], out_hbm.at[pl.ds(base, CH), :],
                              out_sem).wait()

    def compute(b, br):
        inb, rowsb, outb = in_v[b], rows_v[br], out_v[br]

        def group_body(g, tk):
            gb = g * LN
            dv = inb[1, pl.ds(gb, LN)].astype(jnp.float32)
            f0g = inb[2, pl.ds(gb, LN)].astype(jnp.float32)
            f1g = inb[3, pl.ds(gb, LN)].astype(jnp.float32)
            for rr in range(0, LN, 2):
                rows = []
                for r in (rr, rr + 1):
                    i = gb + r
                    rsel = jnp.full((LN,), r, jnp.int32)
                    sb = rowsb[i, pl.ds(D, LN)] * jnp.take(dv, rsel)
                    rows.append((i, sb, jnp.take(f0g, rsel),
                                 jnp.take(f1g, rsel)))
                for j in range(4):
                    for i, sb, f0b, f1b in rows:
                        outb[i, pl.ds(j * LN, LN)] = (
                            (rowsb[i, pl.ds(j * LN, LN)] + sb * wlt[j])
                            + (f0b * w0[j] + f1b * w1[j]))
            return tk

        lax.fori_loop(0, CH // LN, group_body, 0)

    def step(c, b, drain):
        wait_in((b + 1) % NBUF)
        start_gather((b + 1) % NBUF, (b + 1) % 2)
        start_in(c + 2, (b + 2) % NBUF)
        wait_gather((b + 1) % NBUF, b % 2)
        if drain:
            drain_out(b % 2)
        compute(b, b % 2)
        start_out(c, b % 2)

    start_in(0, 0)
    wait_in(0)
    start_gather(0, 0)
    start_in(1, 1)
    for b in range(NBUF):
        step(b, b, b >= 2)

    def outer(p, tk):
        for b in range(NBUF):
            step(p * NBUF + b, b, True)
        return tk

    lax.fori_loop(1, nch // NBUF, outer, 0)

    wait_in(1)
    wait_gather(0, 0)
    for b in range(2):
        drain_out(b)


def _run_sc(pk, ext, wts, n_rows):
    mesh = plsc.VectorSubcoreMesh(core_axis_name="c", subcore_axis_name="s")
    f = pl.kernel(
        _sc_body,
        out_type=jax.ShapeDtypeStruct((n_rows, D), jnp.float32),
        mesh=mesh,
        compiler_params=pltpu.CompilerParams(use_tc_tiling_on_sc=True),
        scratch_types=(
            [pltpu.VMEM((4, CH), jnp.int32) for _ in range(NBUF)]
            + [pltpu.VMEM((CH, W), jnp.float32) for _ in range(2)]
            + [pltpu.VMEM((CH, D), jnp.float32) for _ in range(2)]
            + [pltpu.VMEM((3 * D,), jnp.float32),
               pltpu.SemaphoreType.DMA,
               pltpu.SemaphoreType.DMA,
               pltpu.SemaphoreType.DMA]
        ),
    )
    return f(pk, ext, wts)


def kernel(x, token_table, ac, lt_w, lt_b, df_w, df_b):
    B, L, _ = x.shape
    n = B * L
    acts = x[:, :, 0].reshape(n)
    f0r = x[:, :, 2].reshape(n)
    f1r = x[:, :, 3].reshape(n)
    bias2d = (lt_b + df_b)[None, :]
    wts = jnp.concatenate([lt_w[:, 0], df_w[:, 0], df_w[:, 1]])
    ext = _build_ext_table(token_table, ac, bias2d)
    dif = _build_diff(x[:, :, 1]).reshape(n)
    pk = jnp.stack([acts, dif, f0r, f1r])
    out = _run_sc(pk, ext, wts, n)
    return out.reshape(B, L, D)
